# baseline (speedup 1.0000x reference)
import functools

import jax
import jax.numpy as jnp
import numpy as np
from jax.experimental import pallas as pl

D_MODEL = 768
N_EXPERTS = 64
K = 2
NUM_GATES = 4
NG = NUM_GATES * N_EXPERTS

BT = 4096


def _router_kernel(x_ref, w_ref, s_ref, a_ref,
                   idx_ref, scr_ref, probs_ref, imp_ref, load_ref):
    logits = jax.lax.dot_general(
        x_ref[:], w_ref[:],
        dimension_numbers=(((1,), (1,)), ((), ())),
        preferred_element_type=jnp.float32,
    )
    e = jnp.exp(logits)
    sums = jax.lax.dot_general(
        e, s_ref[:],
        dimension_numbers=(((1,), (0,)), ((), ())),
        preferred_element_type=jnp.float32,
        precision=jax.lax.Precision.HIGHEST,
    )
    pf = e / sums
    probs = jax.lax.dot_general(
        pf, a_ref[:],
        dimension_numbers=(((1,), (0,)), ((), ())),
        preferred_element_type=jnp.float32,
        precision=jax.lax.Precision.HIGHEST,
    )
    probs_ref[:] = probs

    iota = jax.lax.broadcasted_iota(jnp.int32, (BT, N_EXPERTS), 1)
    m1 = jnp.max(probs, axis=-1, keepdims=True)
    i1 = jnp.min(jnp.where(probs == m1, iota, N_EXPERTS), axis=-1, keepdims=True)
    masked = jnp.where(iota == i1, -jnp.inf, probs)
    m2 = jnp.max(masked, axis=-1, keepdims=True)
    i2 = jnp.min(jnp.where(masked == m2, iota, N_EXPERTS), axis=-1, keepdims=True)
    den = jnp.maximum(m1 + m2, 1e-9)
    idx_ref[:] = jnp.concatenate([i1, i2], axis=-1)
    scr_ref[:] = jnp.concatenate([m1 / den, m2 / den], axis=-1)

    @pl.when(pl.program_id(0) == 0)
    def _init():
        imp_ref[:] = jnp.zeros_like(imp_ref)
        load_ref[:] = jnp.zeros_like(load_ref)

    psum = jnp.sum(probs, axis=0, keepdims=True)
    lsum = jnp.sum((probs > 0.0).astype(jnp.float32), axis=0, keepdims=True)
    imp_ref[:] += jnp.broadcast_to(psum, imp_ref.shape)
    load_ref[:] += jnp.broadcast_to(lsum, load_ref.shape)


@functools.partial(jax.jit, static_argnames=())
def kernel(x, W):
    B, S, D = x.shape
    T = B * S
    xf = x.reshape(T, D)
    wf = W.reshape(NG, D)
    s_mat = jnp.asarray(np.kron(np.eye(NUM_GATES, dtype=np.float32),
                                np.ones((N_EXPERTS, N_EXPERTS), np.float32)))
    a_mat = jnp.asarray(np.tile(np.eye(N_EXPERTS, dtype=np.float32),
                                (NUM_GATES, 1)) * (1.0 / NUM_GATES))

    grid = (T // BT,)
    out = pl.pallas_call(
        _router_kernel,
        grid=grid,
        in_specs=[
            pl.BlockSpec((BT, D), lambda i: (i, 0)),
            pl.BlockSpec((NG, D), lambda i: (0, 0)),
            pl.BlockSpec((NG, NG), lambda i: (0, 0)),
            pl.BlockSpec((NG, N_EXPERTS), lambda i: (0, 0)),
        ],
        out_specs=[
            pl.BlockSpec((BT, K), lambda i: (i, 0)),
            pl.BlockSpec((BT, K), lambda i: (i, 0)),
            pl.BlockSpec((BT, N_EXPERTS), lambda i: (i, 0)),
            pl.BlockSpec((8, N_EXPERTS), lambda i: (0, 0)),
            pl.BlockSpec((8, N_EXPERTS), lambda i: (0, 0)),
        ],
        out_shape=[
            jax.ShapeDtypeStruct((T, K), jnp.int32),
            jax.ShapeDtypeStruct((T, K), jnp.float32),
            jax.ShapeDtypeStruct((T, N_EXPERTS), jnp.float32),
            jax.ShapeDtypeStruct((8, N_EXPERTS), jnp.float32),
            jax.ShapeDtypeStruct((8, N_EXPERTS), jnp.float32),
        ],
    )(xf, wf, s_mat, a_mat)
    idx_f, scr_f, probs_f, imp_acc, load_acc = out
    idx = idx_f.reshape(B, S, K)
    scores = scr_f.reshape(B, S, K)
    probs_full = probs_f.reshape(B, S, N_EXPERTS)
    inv_t = 1.0 / float(T)
    importance = imp_acc[0] * inv_t
    load = load_acc[0] * inv_t
    return (idx, scores, probs_full, importance, load)


# speedup vs baseline: 1.0408x; 1.0408x over previous
import functools

import jax
import jax.numpy as jnp
import numpy as np
from jax.experimental import pallas as pl

D_MODEL = 768
N_EXPERTS = 64
K = 2
NUM_GATES = 4
NG = NUM_GATES * N_EXPERTS

BT = 4096


def _router_kernel(x_ref, w_ref,
                   idx_ref, scr_ref, probs_ref, imp_ref, load_ref):
    logits = jax.lax.dot_general(
        x_ref[:], w_ref[:],
        dimension_numbers=(((1,), (1,)), ((), ())),
        preferred_element_type=jnp.float32,
    )
    e = jnp.exp(logits)
    probs = None
    for g in range(NUM_GATES):
        eg = e[:, g * N_EXPERTS:(g + 1) * N_EXPERTS]
        rg = 1.0 / jnp.sum(eg, axis=-1, keepdims=True)
        pg = eg * rg
        probs = pg if probs is None else probs + pg
    probs = probs * (1.0 / NUM_GATES)
    probs_ref[:] = probs

    iota = jax.lax.broadcasted_iota(jnp.int32, (BT, N_EXPERTS), 1)
    m1 = jnp.max(probs, axis=-1, keepdims=True)
    i1 = jnp.min(jnp.where(probs == m1, iota, N_EXPERTS),
                 axis=-1, keepdims=True)
    masked = jnp.where(iota == i1, -jnp.inf, probs)
    m2 = jnp.max(masked, axis=-1, keepdims=True)
    i2 = jnp.min(jnp.where(masked == m2, iota, N_EXPERTS),
                 axis=-1, keepdims=True)
    den = jnp.maximum(m1 + m2, 1e-9)
    idx_ref[:] = jnp.concatenate([i1, i2], axis=-1)
    scr_ref[:] = jnp.concatenate([m1 / den, m2 / den], axis=-1)

    @pl.when(pl.program_id(0) == 0)
    def _init():
        imp_ref[:] = jnp.zeros_like(imp_ref)
        load_ref[:] = jnp.zeros_like(load_ref)

    psum = jnp.sum(probs, axis=0, keepdims=True)
    lsum = jnp.sum((probs > 0.0).astype(jnp.float32), axis=0, keepdims=True)
    imp_ref[:] += jnp.broadcast_to(psum, imp_ref.shape)
    load_ref[:] += jnp.broadcast_to(lsum, load_ref.shape)


@functools.partial(jax.jit, static_argnames=())
def kernel(x, W):
    B, S, D = x.shape
    T = B * S
    xf = x.reshape(T, D)
    wf = W.reshape(NG, D)

    grid = (T // BT,)
    out = pl.pallas_call(
        _router_kernel,
        grid=grid,
        in_specs=[
            pl.BlockSpec((BT, D), lambda i: (i, 0)),
            pl.BlockSpec((NG, D), lambda i: (0, 0)),
        ],
        out_specs=[
            pl.BlockSpec((BT, K), lambda i: (i, 0)),
            pl.BlockSpec((BT, K), lambda i: (i, 0)),
            pl.BlockSpec((BT, N_EXPERTS), lambda i: (i, 0)),
            pl.BlockSpec((8, N_EXPERTS), lambda i: (0, 0)),
            pl.BlockSpec((8, N_EXPERTS), lambda i: (0, 0)),
        ],
        out_shape=[
            jax.ShapeDtypeStruct((T, K), jnp.int32),
            jax.ShapeDtypeStruct((T, K), jnp.float32),
            jax.ShapeDtypeStruct((T, N_EXPERTS), jnp.float32),
            jax.ShapeDtypeStruct((8, N_EXPERTS), jnp.float32),
            jax.ShapeDtypeStruct((8, N_EXPERTS), jnp.float32),
        ],
    )(xf, wf)
    idx_f, scr_f, probs_f, imp_acc, load_acc = out
    idx = idx_f.reshape(B, S, K)
    scores = scr_f.reshape(B, S, K)
    probs_full = probs_f.reshape(B, S, N_EXPERTS)
    inv_t = 1.0 / float(T)
    importance = imp_acc[0] * inv_t
    load = load_acc[0] * inv_t
    return (idx, scores, probs_full, importance, load)


# speedup vs baseline: 1.1088x; 1.0653x over previous
import functools

import jax
import jax.numpy as jnp
import numpy as np
from jax.experimental import pallas as pl

D_MODEL = 768
N_EXPERTS = 64
K = 2
NUM_GATES = 4
NG = NUM_GATES * N_EXPERTS

BT = 4096


def _router_kernel(x_ref, w_ref,
                   idx_ref, scr_ref, probs_ref, imp_ref, load_ref):
    logits = jax.lax.dot_general(
        x_ref[:], w_ref[:],
        dimension_numbers=(((1,), (1,)), ((), ())),
        preferred_element_type=jnp.float32,
    )
    probs = None
    for g in range(NUM_GATES):
        lg = logits[:, g * N_EXPERTS:(g + 1) * N_EXPERTS]
        mg = jnp.max(lg, axis=-1, keepdims=True)
        eg = jnp.exp(lg - mg)
        sg = jnp.sum(eg, axis=-1, keepdims=True)
        pg = eg / sg
        probs = pg if probs is None else probs + pg
    probs = probs * (1.0 / NUM_GATES)
    probs_ref[:] = probs

    iota = jax.lax.broadcasted_iota(jnp.int32, (BT, N_EXPERTS), 1)
    m1 = jnp.max(probs, axis=-1, keepdims=True)
    i1 = jnp.min(jnp.where(probs == m1, iota, N_EXPERTS),
                 axis=-1, keepdims=True)
    masked = jnp.where(iota == i1, -jnp.inf, probs)
    m2 = jnp.max(masked, axis=-1, keepdims=True)
    i2 = jnp.min(jnp.where(masked == m2, iota, N_EXPERTS),
                 axis=-1, keepdims=True)
    den = jnp.maximum(m1 + m2, 1e-9)
    idx_ref[:] = jnp.concatenate([i1, i2], axis=-1)
    scr_ref[:] = jnp.concatenate([m1 / den, m2 / den], axis=-1)

    @pl.when(pl.program_id(0) == 0)
    def _init():
        imp_ref[:] = jnp.zeros_like(imp_ref)
        load_ref[:] = jnp.zeros_like(load_ref)

    psum = jnp.sum(probs, axis=0, keepdims=True)
    lsum = jnp.sum((probs > 0.0).astype(jnp.float32), axis=0, keepdims=True)
    imp_ref[:] += jnp.broadcast_to(psum, imp_ref.shape)
    load_ref[:] += jnp.broadcast_to(lsum, load_ref.shape)


@functools.partial(jax.jit, static_argnames=())
def kernel(x, W):
    B, S, D = x.shape
    T = B * S
    xf = x.reshape(T, D)
    wf = W.reshape(NG, D)

    grid = (T // BT,)
    out = pl.pallas_call(
        _router_kernel,
        grid=grid,
        in_specs=[
            pl.BlockSpec((BT, D), lambda i: (i, 0)),
            pl.BlockSpec((NG, D), lambda i: (0, 0)),
        ],
        out_specs=[
            pl.BlockSpec((BT, K), lambda i: (i, 0)),
            pl.BlockSpec((BT, K), lambda i: (i, 0)),
            pl.BlockSpec((BT, N_EXPERTS), lambda i: (i, 0)),
            pl.BlockSpec((8, N_EXPERTS), lambda i: (0, 0)),
            pl.BlockSpec((8, N_EXPERTS), lambda i: (0, 0)),
        ],
        out_shape=[
            jax.ShapeDtypeStruct((T, K), jnp.int32),
            jax.ShapeDtypeStruct((T, K), jnp.float32),
            jax.ShapeDtypeStruct((T, N_EXPERTS), jnp.float32),
            jax.ShapeDtypeStruct((8, N_EXPERTS), jnp.float32),
            jax.ShapeDtypeStruct((8, N_EXPERTS), jnp.float32),
        ],
    )(xf, wf)
    idx_f, scr_f, probs_f, imp_acc, load_acc = out
    idx = idx_f.reshape(B, S, K)
    scores = scr_f.reshape(B, S, K)
    probs_full = probs_f.reshape(B, S, N_EXPERTS)
    inv_t = 1.0 / float(T)
    importance = imp_acc[0] * inv_t
    load = load_acc[0] * inv_t
    return (idx, scores, probs_full, importance, load)


# speedup vs baseline: 1.2222x; 1.1023x over previous
import functools

import jax
import jax.numpy as jnp
from jax.experimental import pallas as pl

D_MODEL = 768
N_EXPERTS = 64
K = 2
NUM_GATES = 4
NG = NUM_GATES * N_EXPERTS

BT = 4096


def _router_kernel(x_ref, w_ref,
                   idx_ref, scr_ref, probs_ref, imp_ref, load_ref):
    logits = jax.lax.dot_general(
        x_ref[0], w_ref[:],
        dimension_numbers=(((1,), (1,)), ((), ())),
        preferred_element_type=jnp.float32,
    )
    probs = None
    for g in range(NUM_GATES):
        lg = logits[:, g * N_EXPERTS:(g + 1) * N_EXPERTS]
        mg = jnp.max(lg, axis=-1, keepdims=True)
        eg = jnp.exp(lg - mg)
        sg = jnp.sum(eg, axis=-1, keepdims=True)
        pg = eg / sg
        probs = pg if probs is None else probs + pg
    probs = probs * (1.0 / NUM_GATES)
    probs_ref[0] = probs

    iota = jax.lax.broadcasted_iota(
        jnp.int32, (BT, N_EXPERTS), 1).astype(jnp.float32)
    m1 = jnp.max(probs, axis=-1, keepdims=True)
    i1 = jnp.min(jnp.where(probs == m1, iota, float(N_EXPERTS)),
                 axis=-1, keepdims=True)
    masked = jnp.where(iota == i1, -jnp.inf, probs)
    m2 = jnp.max(masked, axis=-1, keepdims=True)
    i2 = jnp.min(jnp.where(masked == m2, iota, float(N_EXPERTS)),
                 axis=-1, keepdims=True)
    den = jnp.maximum(m1 + m2, 1e-9)
    idx_ref[0] = jnp.concatenate([i1, i2], axis=-1).astype(jnp.int32)
    scr_ref[0] = jnp.concatenate([m1 / den, m2 / den], axis=-1)

    @pl.when((pl.program_id(0) == 0) & (pl.program_id(1) == 0))
    def _init():
        imp_ref[:] = jnp.zeros_like(imp_ref)
        load_ref[:] = jnp.zeros_like(load_ref)

    psum = jnp.sum(probs, axis=0, keepdims=True)
    lsum = jnp.sum((probs > 0.0).astype(jnp.float32), axis=0, keepdims=True)
    imp_ref[:] += jnp.broadcast_to(psum, imp_ref.shape)
    load_ref[:] += jnp.broadcast_to(lsum, load_ref.shape)


@functools.partial(jax.jit, static_argnames=())
def kernel(x, W):
    B, S, D = x.shape
    T = B * S
    wf = W.reshape(NG, D)

    grid = (B, S // BT)
    out = pl.pallas_call(
        _router_kernel,
        grid=grid,
        in_specs=[
            pl.BlockSpec((1, BT, D), lambda b, i: (b, i, 0)),
            pl.BlockSpec((NG, D), lambda b, i: (0, 0)),
        ],
        out_specs=[
            pl.BlockSpec((1, BT, K), lambda b, i: (b, i, 0)),
            pl.BlockSpec((1, BT, K), lambda b, i: (b, i, 0)),
            pl.BlockSpec((1, BT, N_EXPERTS), lambda b, i: (b, i, 0)),
            pl.BlockSpec((8, N_EXPERTS), lambda b, i: (0, 0)),
            pl.BlockSpec((8, N_EXPERTS), lambda b, i: (0, 0)),
        ],
        out_shape=[
            jax.ShapeDtypeStruct((B, S, K), jnp.int32),
            jax.ShapeDtypeStruct((B, S, K), jnp.float32),
            jax.ShapeDtypeStruct((B, S, N_EXPERTS), jnp.float32),
            jax.ShapeDtypeStruct((8, N_EXPERTS), jnp.float32),
            jax.ShapeDtypeStruct((8, N_EXPERTS), jnp.float32),
        ],
    )(x, wf)
    idx, scores, probs_full, imp_acc, load_acc = out
    inv_t = 1.0 / float(T)
    importance = imp_acc[0] * inv_t
    load = load_acc[0] * inv_t
    return (idx, scores, probs_full, importance, load)


# speedup vs baseline: 1.2268x; 1.0037x over previous
import functools

import jax
import jax.numpy as jnp
from jax.experimental import pallas as pl

D_MODEL = 768
N_EXPERTS = 64
K = 2
NUM_GATES = 4
NG = NUM_GATES * N_EXPERTS

BT = 4096


def _router_kernel(x_ref, w_ref,
                   idx_ref, scr_ref, probs_ref, imp_ref, load_ref):
    logits = jax.lax.dot_general(
        x_ref[0], w_ref[:],
        dimension_numbers=(((1,), (0,)), ((), ())),
        preferred_element_type=jnp.float32,
    )
    probs = None
    for g in range(NUM_GATES):
        lg = logits[:, g * N_EXPERTS:(g + 1) * N_EXPERTS]
        mg = jnp.max(lg, axis=-1, keepdims=True)
        eg = jnp.exp(lg - mg)
        sg = jnp.sum(eg, axis=-1, keepdims=True)
        pg = eg / sg
        probs = pg if probs is None else probs + pg
    probs = probs * (1.0 / NUM_GATES)
    probs_ref[0] = probs

    iota = jax.lax.broadcasted_iota(
        jnp.int32, (BT, N_EXPERTS), 1).astype(jnp.float32)
    m1 = jnp.max(probs, axis=-1, keepdims=True)
    i1 = jnp.min(jnp.where(probs == m1, iota, float(N_EXPERTS)),
                 axis=-1, keepdims=True)
    masked = jnp.where(iota == i1, -jnp.inf, probs)
    m2 = jnp.max(masked, axis=-1, keepdims=True)
    i2 = jnp.min(jnp.where(masked == m2, iota, float(N_EXPERTS)),
                 axis=-1, keepdims=True)
    den = jnp.maximum(m1 + m2, 1e-9)
    idx_ref[0] = jnp.concatenate([i1, i2], axis=-1).astype(jnp.int32)
    scr_ref[0] = jnp.concatenate([m1 / den, m2 / den], axis=-1)

    @pl.when((pl.program_id(0) == 0) & (pl.program_id(1) == 0))
    def _init():
        imp_ref[:] = jnp.zeros_like(imp_ref)
        load_ref[:] = jnp.zeros_like(load_ref)

    psum = jnp.sum(probs, axis=0, keepdims=True)
    lsum = jnp.sum((probs > 0.0).astype(jnp.float32), axis=0, keepdims=True)
    imp_ref[:] += jnp.broadcast_to(psum, imp_ref.shape)
    load_ref[:] += jnp.broadcast_to(lsum, load_ref.shape)


@functools.partial(jax.jit, static_argnames=())
def kernel(x, W):
    B, S, D = x.shape
    T = B * S
    wt = W.reshape(NG, D).T

    grid = (B, S // BT)
    out = pl.pallas_call(
        _router_kernel,
        grid=grid,
        in_specs=[
            pl.BlockSpec((1, BT, D), lambda b, i: (b, i, 0)),
            pl.BlockSpec((D, NG), lambda b, i: (0, 0)),
        ],
        out_specs=[
            pl.BlockSpec((1, BT, K), lambda b, i: (b, i, 0)),
            pl.BlockSpec((1, BT, K), lambda b, i: (b, i, 0)),
            pl.BlockSpec((1, BT, N_EXPERTS), lambda b, i: (b, i, 0)),
            pl.BlockSpec((8, N_EXPERTS), lambda b, i: (0, 0)),
            pl.BlockSpec((8, N_EXPERTS), lambda b, i: (0, 0)),
        ],
        out_shape=[
            jax.ShapeDtypeStruct((B, S, K), jnp.int32),
            jax.ShapeDtypeStruct((B, S, K), jnp.float32),
            jax.ShapeDtypeStruct((B, S, N_EXPERTS), jnp.float32),
            jax.ShapeDtypeStruct((8, N_EXPERTS), jnp.float32),
            jax.ShapeDtypeStruct((8, N_EXPERTS), jnp.float32),
        ],
    )(x, wt)
    idx, scores, probs_full, imp_acc, load_acc = out
    inv_t = 1.0 / float(T)
    importance = imp_acc[0] * inv_t
    load = load_acc[0] * inv_t
    return (idx, scores, probs_full, importance, load)


# speedup vs baseline: 1.2533x; 1.0216x over previous
import functools

import jax
import jax.numpy as jnp
from jax.experimental import pallas as pl
from jax.experimental.pallas import tpu as pltpu

D_MODEL = 768
N_EXPERTS = 64
K = 2
NUM_GATES = 4
NG = NUM_GATES * N_EXPERTS

BT = 4096


def _router_kernel(x_ref, w_ref,
                   idx_ref, scr_ref, probs_ref, imp_ref, load_ref):
    logits = jax.lax.dot_general(
        x_ref[0], w_ref[:],
        dimension_numbers=(((1,), (0,)), ((), ())),
        preferred_element_type=jnp.float32,
    )
    probs = None
    for g in range(NUM_GATES):
        lg = logits[:, g * N_EXPERTS:(g + 1) * N_EXPERTS]
        mg = jnp.max(lg, axis=-1, keepdims=True)
        eg = jnp.exp(lg - mg)
        sg = jnp.sum(eg, axis=-1, keepdims=True)
        pg = eg / sg
        probs = pg if probs is None else probs + pg
    probs = probs * (1.0 / NUM_GATES)
    probs_ref[0] = probs

    iota = jax.lax.broadcasted_iota(
        jnp.int32, (BT, N_EXPERTS), 1).astype(jnp.float32)
    m1 = jnp.max(probs, axis=-1, keepdims=True)
    i1 = jnp.min(jnp.where(probs == m1, iota, float(N_EXPERTS)),
                 axis=-1, keepdims=True)
    masked = jnp.where(iota == i1, -jnp.inf, probs)
    m2 = jnp.max(masked, axis=-1, keepdims=True)
    i2 = jnp.min(jnp.where(masked == m2, iota, float(N_EXPERTS)),
                 axis=-1, keepdims=True)
    den = jnp.maximum(m1 + m2, 1e-9)
    idx_ref[0] = jnp.concatenate([i1, i2], axis=-1).astype(jnp.int32)
    scr_ref[0] = jnp.concatenate([m1 / den, m2 / den], axis=-1)

    psum = jnp.sum(probs, axis=0, keepdims=True)
    lsum = jnp.sum((probs > 0.0).astype(jnp.float32), axis=0, keepdims=True)
    imp_ref[0] = jnp.broadcast_to(psum, imp_ref.shape[1:])
    load_ref[0] = jnp.broadcast_to(lsum, load_ref.shape[1:])


@functools.partial(jax.jit, static_argnames=())
def kernel(x, W):
    B, S, D = x.shape
    T = B * S
    wt = W.reshape(NG, D).T

    grid = (B, S // BT)
    out = pl.pallas_call(
        _router_kernel,
        grid=grid,
        in_specs=[
            pl.BlockSpec((1, BT, D), lambda b, i: (b, i, 0)),
            pl.BlockSpec((D, NG), lambda b, i: (0, 0)),
        ],
        out_specs=[
            pl.BlockSpec((1, BT, K), lambda b, i: (b, i, 0)),
            pl.BlockSpec((1, BT, K), lambda b, i: (b, i, 0)),
            pl.BlockSpec((1, BT, N_EXPERTS), lambda b, i: (b, i, 0)),
            pl.BlockSpec((1, 8, N_EXPERTS),
                         lambda b, i: (b * (S // BT) + i, 0, 0)),
            pl.BlockSpec((1, 8, N_EXPERTS),
                         lambda b, i: (b * (S // BT) + i, 0, 0)),
        ],
        out_shape=[
            jax.ShapeDtypeStruct((B, S, K), jnp.int32),
            jax.ShapeDtypeStruct((B, S, K), jnp.float32),
            jax.ShapeDtypeStruct((B, S, N_EXPERTS), jnp.float32),
            jax.ShapeDtypeStruct((T // BT, 8, N_EXPERTS), jnp.float32),
            jax.ShapeDtypeStruct((T // BT, 8, N_EXPERTS), jnp.float32),
        ],
        compiler_params=pltpu.CompilerParams(
            dimension_semantics=("parallel", "parallel")),
    )(x, wt)
    idx, scores, probs_full, imp_acc, load_acc = out
    inv_t = 1.0 / float(T)
    importance = jnp.sum(imp_acc[:, 0], axis=0) * inv_t
    load = jnp.sum(load_acc[:, 0], axis=0) * inv_t
    return (idx, scores, probs_full, importance, load)
